# baseline (speedup 1.0000x reference)
import functools

import jax
import jax.numpy as jnp
from jax import lax
from jax.experimental import pallas as pl
from jax.experimental.pallas import tpu as pltpu
from jax.experimental.pallas import tpu_sc as plsc

_N = 10000
_E = 160000
_D_IN = 256
_D_H = 512

_NC, _NS = 2, 16
_NP = 10240
_CH = 128
_EP = 163840
_ER = _EP // _CH

@functools.cache
def _mesh():
    return plsc.VectorSubcoreMesh(core_axis_name="c", subcore_axis_name="s",
                                  num_cores=_NC, num_subcores=_NS)


@functools.cache
def _make_deg_kernel():
    return functools.partial(
        pl.kernel,
        out_type=jax.ShapeDtypeStruct((32, _NP * 8), jnp.float32),
        mesh=_mesh(),
        compiler_params=pltpu.CompilerParams(needs_layout_passes=False),
        scratch_types=[
            pltpu.VMEM((_ER // 32, _CH), jnp.int32),
            pltpu.VMEM((_NP * 8,), jnp.float32),
        ],
    )(_deg_body)


def _deg_body(dst_hbm, zeros_hbm, out_hbm, idx_v, acc_v):
    c = lax.axis_index("c")
    s = lax.axis_index("s")
    w = c * _NS + s
    pltpu.sync_copy(zeros_hbm, acc_v)
    pltpu.sync_copy(dst_hbm.at[pl.ds(w * 40, 40)], idx_v)
    ones16 = jnp.ones((16,), jnp.float32)

    def row(i, _):
        for j in range(8):
            d16 = idx_v[i, pl.ds(j * 16, 16)]
            plsc.addupdate_scatter(acc_v, [d16 * 8], ones16)
        return 0

    lax.fori_loop(0, 40, row, 0)
    pltpu.sync_copy(acc_v, out_hbm.at[w])


@functools.cache
def _make_agg_kernel(n_sides: int):
    ngpt = n_sides * 16 // 32
    SROWS = 8
    SEC = _ER // SROWS

    @functools.partial(
        pl.kernel,
        out_type=jax.ShapeDtypeStruct((32 * ngpt, _NP * 8), jnp.float32),
        mesh=_mesh(),
        compiler_params=pltpu.CompilerParams(needs_layout_passes=False),
        scratch_types=[
            pltpu.VMEM((2 * SROWS, _CH), jnp.int32),
            pltpu.VMEM((2 * SROWS, _CH), jnp.int32),
            pltpu.VMEM((_NP * 4,), jnp.int32),
            pltpu.VMEM((_NP * 8,), jnp.float32),
            pltpu.SemaphoreType.DMA,
            pltpu.SemaphoreType.DMA,
        ],
    )
    def _agg(edges_hbm, feat_hbm, zeros_hbm, out_hbm,
             b0, b1, feat_v, acc_v, sem0, sem1):
        c = lax.axis_index("c")
        s = lax.axis_index("s")
        w = c * _NS + s
        himask = jnp.full((16,), -65536, jnp.int32)

        def compute(buf):
            for i in range(SROWS):
                for k in range(8):
                    sl = pl.ds(k * 16, 16)
                    s4 = buf[2 * i, sl] * 4
                    d8 = buf[2 * i + 1, sl] * 8
                    for jj in range(4):
                        vals = plsc.load_gather(feat_v, [s4 + jj])
                        lo = plsc.bitcast(lax.shift_left(vals, 16),
                                          jnp.float32)
                        hi = plsc.bitcast(lax.bitwise_and(vals, himask),
                                          jnp.float32)
                        plsc.addupdate_scatter(acc_v, [d8 + (2 * jj)], lo)
                        plsc.addupdate_scatter(acc_v, [d8 + (2 * jj + 1)], hi)

        for gl in range(ngpt):
            g = w + 32 * gl
            pltpu.sync_copy(feat_hbm.at[g], feat_v)
            pltpu.sync_copy(zeros_hbm, acc_v)

            def body(sec, _):
                pltpu.sync_copy(
                    edges_hbm.at[pl.ds(sec * 2 * SROWS, 2 * SROWS)], b0)
                compute(b0)
                return 0

            lax.fori_loop(0, SEC, body, 0)
            pltpu.sync_copy(acc_v, out_hbm.at[g])

    return _agg


def _pack_groups(x, n_sides):
    t = x.reshape(n_sides, _NP, 16, 8).transpose(0, 2, 1, 3)
    t = t.astype(jnp.bfloat16).reshape(n_sides * 16, _NP, 4, 2)
    return lax.bitcast_convert_type(t, jnp.int32).reshape(n_sides * 16, _NP * 4)


def _combine_body(cw1z, cw1h, l1z, l1h, cw2z, cw2h, l2z, l2h,
                  cb1z, cb1h, lb1z, lb1h, cb2z, cb2h, lb2z, lb2h,
                  wz1, wh1, wz2, wh2, bz1, bh1, bz2, bh2):
    f32 = jnp.float32
    wz1[...] = jnp.dot(cw1z[...], l1z[...], preferred_element_type=f32)
    wh1[...] = jnp.dot(cw1h[...], l1h[...], preferred_element_type=f32)
    wz2[...] = jnp.dot(cw2z[...], l2z[...], preferred_element_type=f32)
    wh2[...] = jnp.dot(cw2h[...], l2h[...], preferred_element_type=f32)
    bz1[...] = jnp.dot(cb1z[...], l1z[...], preferred_element_type=f32) + lb1z[...]
    bh1[...] = jnp.dot(cb1h[...], l1h[...], preferred_element_type=f32) + lb1h[...]
    bz2[...] = jnp.dot(cb2z[...], l2z[...], preferred_element_type=f32) + lb2z[...]
    bh2[...] = jnp.dot(cb2h[...], l2h[...], preferred_element_type=f32) + lb2h[...]


def _combine_weights(conv1_W, conv1_b, lin1_W, lin1_b,
                     conv2_W, conv2_b, lin2_W, lin2_b):
    f32 = jnp.float32
    outs = [
        jax.ShapeDtypeStruct((_D_IN, _D_H), f32),
        jax.ShapeDtypeStruct((_D_IN, _D_H), f32),
        jax.ShapeDtypeStruct((_D_H, _D_H), f32),
        jax.ShapeDtypeStruct((_D_H, _D_H), f32),
        jax.ShapeDtypeStruct((1, _D_H), f32),
        jax.ShapeDtypeStruct((1, _D_H), f32),
        jax.ShapeDtypeStruct((1, _D_H), f32),
        jax.ShapeDtypeStruct((1, _D_H), f32),
    ]
    return pl.pallas_call(_combine_body, out_shape=outs)(
        conv1_W[0], conv1_W[2], lin1_W[0][:_D_H], lin1_W[2][:_D_H],
        conv2_W[0], conv2_W[2], lin2_W[0][:_D_H], lin2_W[2][:_D_H],
        conv1_b[0][None], conv1_b[2][None], lin1_b[0][None], lin1_b[2][None],
        conv2_b[0][None], conv2_b[2][None], lin2_b[0][None], lin2_b[2][None],
    )


def _scale_body(degp, xblk, disb, xs):
    deg = jnp.sum(degp[:, :, 0], axis=0) + 1.0
    dis = lax.rsqrt(deg)
    disb[...] = jnp.broadcast_to(dis[:, None], (256, 128))
    xsv = xblk[...] * dis[:, None]
    xs[0] = xsv[:, :128]
    xs[1] = xsv[:, 128:]


def _tc_scale(deg_part, x_p):
    grid = (_NP // 256,)
    return pl.pallas_call(
        _scale_body,
        grid=grid,
        in_specs=[
            pl.BlockSpec((32, 256, 8), lambda i: (0, i, 0)),
            pl.BlockSpec((256, _D_IN), lambda i: (i, 0)),
        ],
        out_specs=[
            pl.BlockSpec((256, 128), lambda i: (i, 0)),
            pl.BlockSpec((2, 256, 128), lambda i: (0, i, 0)),
        ],
        out_shape=[
            jax.ShapeDtypeStruct((_NP, 128), jnp.float32),
            jax.ShapeDtypeStruct((2, _NP, 128), jnp.float32),
        ],
    )(deg_part, x_p)


def _layer1_body(agg, xs, disb, wz, wh, bz, bh, hs):
    d = disb[...]
    xa = jnp.concatenate([d * (agg[0] + xs[0]), d * (agg[1] + xs[1])], axis=1)
    z = jax.nn.sigmoid(jnp.dot(xa, wz[...], preferred_element_type=jnp.float32) + bz[...])
    ht = jnp.tanh(jnp.dot(xa, wh[...], preferred_element_type=jnp.float32) + bh[...])
    h1 = jax.nn.relu((1.0 - z) * ht)
    for c in range(4):
        hs[c] = h1[:, c * 128:(c + 1) * 128] * d


def _tc_layer1(agg1, xs, dis_b, wz, wh, bz, bh):
    grid = (_NP // 256,)
    return pl.pallas_call(
        _layer1_body,
        grid=grid,
        in_specs=[
            pl.BlockSpec((2, 256, 128), lambda i: (0, i, 0)),
            pl.BlockSpec((2, 256, 128), lambda i: (0, i, 0)),
            pl.BlockSpec((256, 128), lambda i: (i, 0)),
            pl.BlockSpec((_D_IN, _D_H), lambda i: (0, 0)),
            pl.BlockSpec((_D_IN, _D_H), lambda i: (0, 0)),
            pl.BlockSpec((1, _D_H), lambda i: (0, 0)),
            pl.BlockSpec((1, _D_H), lambda i: (0, 0)),
        ],
        out_specs=pl.BlockSpec((4, 256, 128), lambda i: (0, i, 0)),
        out_shape=jax.ShapeDtypeStruct((4, _NP, 128), jnp.float32),
    )(agg1.reshape(2, _NP, 128), xs, dis_b, wz, wh, bz, bh)


def _layer2_body(agg, hsin, disb, wz, wh, bz, bh, out):
    d = disb[...]
    ha = jnp.concatenate([d * (agg[c] + hsin[c]) for c in range(4)], axis=1)
    z = jax.nn.sigmoid(jnp.dot(ha, wz[...], preferred_element_type=jnp.float32) + bz[...])
    ht = jnp.tanh(jnp.dot(ha, wh[...], preferred_element_type=jnp.float32) + bh[...])
    out[...] = (1.0 - z) * ht


def _tc_layer2(agg2, hs, dis_b, wz, wh, bz, bh):
    grid = (_NP // 256,)
    return pl.pallas_call(
        _layer2_body,
        grid=grid,
        in_specs=[
            pl.BlockSpec((4, 256, 128), lambda i: (0, i, 0)),
            pl.BlockSpec((4, 256, 128), lambda i: (0, i, 0)),
            pl.BlockSpec((256, 128), lambda i: (i, 0)),
            pl.BlockSpec((_D_H, _D_H), lambda i: (0, 0)),
            pl.BlockSpec((_D_H, _D_H), lambda i: (0, 0)),
            pl.BlockSpec((1, _D_H), lambda i: (0, 0)),
            pl.BlockSpec((1, _D_H), lambda i: (0, 0)),
        ],
        out_specs=pl.BlockSpec((256, _D_H), lambda i: (i, 0)),
        out_shape=jax.ShapeDtypeStruct((_NP, _D_H), jnp.float32),
    )(agg2.reshape(4, _NP, 128), hs, dis_b, wz, wh, bz, bh)


def kernel(x, edge_index, conv1_W, conv1_b, lin1_W, lin1_b,
           conv2_W, conv2_b, lin2_W, lin2_b):
    src = edge_index[0]
    dst = edge_index[1]
    pad = _EP - _E
    src_p = jnp.concatenate([src, jnp.zeros((pad,), jnp.int32)]).reshape(_ER, _CH)
    dst_p = jnp.concatenate([dst, jnp.full((pad,), _N, jnp.int32)]).reshape(_ER, _CH)
    x_p = jnp.pad(x, ((0, _NP - _N), (0, 0)))

    wz1, wh1, wz2, wh2, bz1, bh1, bz2, bh2 = _combine_weights(
        conv1_W, conv1_b, lin1_W, lin1_b, conv2_W, conv2_b, lin2_W, lin2_b)

    zeros_c = jnp.zeros((_NP * 8,), jnp.float32)
    deg_part = _make_deg_kernel()(dst_p, zeros_c).reshape(32, _NP, 8)
    dis_b, xs = _tc_scale(deg_part, x_p)

    edges_il = jnp.stack([src_p, dst_p], axis=1).reshape(2 * _ER, _CH)

    agg1_raw = _make_agg_kernel(2)(edges_il, _pack_groups(xs, 2), zeros_c)
    agg1 = agg1_raw.reshape(2, 16, _NP, 8).transpose(0, 2, 1, 3).reshape(2 * _NP, 128)
    hs = _tc_layer1(agg1, xs, dis_b, wz1, wh1, bz1, bh1)

    agg2_raw = _make_agg_kernel(4)(edges_il, _pack_groups(hs, 4), zeros_c)
    agg2 = agg2_raw.reshape(4, 16, _NP, 8).transpose(0, 2, 1, 3).reshape(4 * _NP, 128)
    out_full = _tc_layer2(agg2, hs, dis_b, wz2, wh2, bz2, bh2)
    return out_full[:_N]


# speedup vs baseline: 1.0837x; 1.0837x over previous
import functools

import jax
import jax.numpy as jnp
from jax import lax
from jax.experimental import pallas as pl
from jax.experimental.pallas import tpu as pltpu
from jax.experimental.pallas import tpu_sc as plsc

_N = 10000
_E = 160000
_D_IN = 256
_D_H = 512

_NC, _NS = 2, 16
_NP = 10240
_CH = 128
_EP = 163840
_ER = _EP // _CH

@functools.cache
def _mesh():
    return plsc.VectorSubcoreMesh(core_axis_name="c", subcore_axis_name="s",
                                  num_cores=_NC, num_subcores=_NS)


@functools.cache
def _make_deg_kernel():
    return functools.partial(
        pl.kernel,
        out_type=jax.ShapeDtypeStruct((32, _NP * 8), jnp.float32),
        mesh=_mesh(),
        compiler_params=pltpu.CompilerParams(needs_layout_passes=False),
        scratch_types=[
            pltpu.VMEM((_ER // 32, _CH), jnp.int32),
            pltpu.VMEM((_NP * 8,), jnp.float32),
        ],
    )(_deg_body)


def _deg_body(dst_hbm, zeros_hbm, out_hbm, idx_v, acc_v):
    c = lax.axis_index("c")
    s = lax.axis_index("s")
    w = c * _NS + s
    pltpu.sync_copy(zeros_hbm, acc_v)
    pltpu.sync_copy(dst_hbm.at[pl.ds(w * 40, 40)], idx_v)
    ones16 = jnp.ones((16,), jnp.float32)

    def row(i, _):
        for j in range(8):
            d16 = idx_v[i, pl.ds(j * 16, 16)]
            plsc.addupdate_scatter(acc_v, [d16 * 8], ones16)
        return 0

    lax.fori_loop(0, 40, row, 0)
    pltpu.sync_copy(acc_v, out_hbm.at[w])


@functools.cache
def _make_agg_kernel(n_sides: int):
    ngpt = n_sides * 16 // 32
    SROWS = 8
    SEC = _ER // SROWS

    @functools.partial(
        pl.kernel,
        out_type=jax.ShapeDtypeStruct((32 * ngpt, _NP * 8), jnp.float32),
        mesh=_mesh(),
        compiler_params=pltpu.CompilerParams(needs_layout_passes=False),
        scratch_types=[
            pltpu.VMEM((2 * SROWS, _CH), jnp.int32),
            pltpu.VMEM((2 * SROWS, _CH), jnp.int32),
            pltpu.VMEM((_NP * 4,), jnp.int32),
            pltpu.VMEM((_NP * 8,), jnp.float32),
            pltpu.SemaphoreType.DMA,
            pltpu.SemaphoreType.DMA,
        ],
    )
    def _agg(edges_hbm, feat_hbm, zeros_hbm, out_hbm,
             b0, b1, feat_v, acc_v, sem0, sem1):
        c = lax.axis_index("c")
        s = lax.axis_index("s")
        w = c * _NS + s
        himask = jnp.full((16,), -65536, jnp.int32)
        iota16 = lax.iota(jnp.int32, 16)
        cp4 = iota16 % 4
        cp4x2 = cp4 * 2
        cp4x2p1 = cp4x2 + 1
        quad_pats = [iota16 // 4 + 4 * q for q in range(4)]

        def compute(buf):
            for i in range(SROWS):
                for k in range(8):
                    sl = pl.ds(k * 16, 16)
                    s4 = buf[2 * i, sl] * 4
                    d8 = buf[2 * i + 1, sl] * 8
                    for q in range(4):
                        pat = quad_pats[q]
                        squad = s4.at[pat].get(mode="promise_in_bounds")
                        dquad = d8.at[pat].get(mode="promise_in_bounds")
                        vals = plsc.load_gather(feat_v, [squad + cp4])
                        lo = plsc.bitcast(lax.shift_left(vals, 16),
                                          jnp.float32)
                        hi = plsc.bitcast(lax.bitwise_and(vals, himask),
                                          jnp.float32)
                        plsc.addupdate_scatter(acc_v, [dquad + cp4x2], lo)
                        plsc.addupdate_scatter(acc_v, [dquad + cp4x2p1], hi)

        for gl in range(ngpt):
            g = w + 32 * gl
            pltpu.sync_copy(feat_hbm.at[g], feat_v)
            pltpu.sync_copy(zeros_hbm, acc_v)

            def body(sec, _):
                pltpu.sync_copy(
                    edges_hbm.at[pl.ds(sec * 2 * SROWS, 2 * SROWS)], b0)
                compute(b0)
                return 0

            lax.fori_loop(0, SEC, body, 0)
            pltpu.sync_copy(acc_v, out_hbm.at[g])

    return _agg


def _pack_groups(x, n_sides):
    t = x.reshape(n_sides, _NP, 16, 8).transpose(0, 2, 1, 3)
    t = t.astype(jnp.bfloat16).reshape(n_sides * 16, _NP, 4, 2)
    return lax.bitcast_convert_type(t, jnp.int32).reshape(n_sides * 16, _NP * 4)


def _combine_body(cw1z, cw1h, l1z, l1h, cw2z, cw2h, l2z, l2h,
                  cb1z, cb1h, lb1z, lb1h, cb2z, cb2h, lb2z, lb2h,
                  wz1, wh1, wz2, wh2, bz1, bh1, bz2, bh2):
    f32 = jnp.float32
    wz1[...] = jnp.dot(cw1z[...], l1z[...], preferred_element_type=f32)
    wh1[...] = jnp.dot(cw1h[...], l1h[...], preferred_element_type=f32)
    wz2[...] = jnp.dot(cw2z[...], l2z[...], preferred_element_type=f32)
    wh2[...] = jnp.dot(cw2h[...], l2h[...], preferred_element_type=f32)
    bz1[...] = jnp.dot(cb1z[...], l1z[...], preferred_element_type=f32) + lb1z[...]
    bh1[...] = jnp.dot(cb1h[...], l1h[...], preferred_element_type=f32) + lb1h[...]
    bz2[...] = jnp.dot(cb2z[...], l2z[...], preferred_element_type=f32) + lb2z[...]
    bh2[...] = jnp.dot(cb2h[...], l2h[...], preferred_element_type=f32) + lb2h[...]


def _combine_weights(conv1_W, conv1_b, lin1_W, lin1_b,
                     conv2_W, conv2_b, lin2_W, lin2_b):
    f32 = jnp.float32
    outs = [
        jax.ShapeDtypeStruct((_D_IN, _D_H), f32),
        jax.ShapeDtypeStruct((_D_IN, _D_H), f32),
        jax.ShapeDtypeStruct((_D_H, _D_H), f32),
        jax.ShapeDtypeStruct((_D_H, _D_H), f32),
        jax.ShapeDtypeStruct((1, _D_H), f32),
        jax.ShapeDtypeStruct((1, _D_H), f32),
        jax.ShapeDtypeStruct((1, _D_H), f32),
        jax.ShapeDtypeStruct((1, _D_H), f32),
    ]
    return pl.pallas_call(_combine_body, out_shape=outs)(
        conv1_W[0], conv1_W[2], lin1_W[0][:_D_H], lin1_W[2][:_D_H],
        conv2_W[0], conv2_W[2], lin2_W[0][:_D_H], lin2_W[2][:_D_H],
        conv1_b[0][None], conv1_b[2][None], lin1_b[0][None], lin1_b[2][None],
        conv2_b[0][None], conv2_b[2][None], lin2_b[0][None], lin2_b[2][None],
    )


def _scale_body(degp, xblk, disb, xs):
    deg = jnp.sum(degp[:, :, 0], axis=0) + 1.0
    dis = lax.rsqrt(deg)
    disb[...] = jnp.broadcast_to(dis[:, None], (256, 128))
    xsv = xblk[...] * dis[:, None]
    xs[0] = xsv[:, :128]
    xs[1] = xsv[:, 128:]


def _tc_scale(deg_part, x_p):
    grid = (_NP // 256,)
    return pl.pallas_call(
        _scale_body,
        grid=grid,
        in_specs=[
            pl.BlockSpec((32, 256, 8), lambda i: (0, i, 0)),
            pl.BlockSpec((256, _D_IN), lambda i: (i, 0)),
        ],
        out_specs=[
            pl.BlockSpec((256, 128), lambda i: (i, 0)),
            pl.BlockSpec((2, 256, 128), lambda i: (0, i, 0)),
        ],
        out_shape=[
            jax.ShapeDtypeStruct((_NP, 128), jnp.float32),
            jax.ShapeDtypeStruct((2, _NP, 128), jnp.float32),
        ],
    )(deg_part, x_p)


def _layer1_body(agg, xs, disb, wz, wh, bz, bh, hs):
    d = disb[...]
    xa = jnp.concatenate([d * (agg[0] + xs[0]), d * (agg[1] + xs[1])], axis=1)
    z = jax.nn.sigmoid(jnp.dot(xa, wz[...], preferred_element_type=jnp.float32) + bz[...])
    ht = jnp.tanh(jnp.dot(xa, wh[...], preferred_element_type=jnp.float32) + bh[...])
    h1 = jax.nn.relu((1.0 - z) * ht)
    for c in range(4):
        hs[c] = h1[:, c * 128:(c + 1) * 128] * d


def _tc_layer1(agg1, xs, dis_b, wz, wh, bz, bh):
    grid = (_NP // 256,)
    return pl.pallas_call(
        _layer1_body,
        grid=grid,
        in_specs=[
            pl.BlockSpec((2, 256, 128), lambda i: (0, i, 0)),
            pl.BlockSpec((2, 256, 128), lambda i: (0, i, 0)),
            pl.BlockSpec((256, 128), lambda i: (i, 0)),
            pl.BlockSpec((_D_IN, _D_H), lambda i: (0, 0)),
            pl.BlockSpec((_D_IN, _D_H), lambda i: (0, 0)),
            pl.BlockSpec((1, _D_H), lambda i: (0, 0)),
            pl.BlockSpec((1, _D_H), lambda i: (0, 0)),
        ],
        out_specs=pl.BlockSpec((4, 256, 128), lambda i: (0, i, 0)),
        out_shape=jax.ShapeDtypeStruct((4, _NP, 128), jnp.float32),
    )(agg1.reshape(2, _NP, 128), xs, dis_b, wz, wh, bz, bh)


def _layer2_body(agg, hsin, disb, wz, wh, bz, bh, out):
    d = disb[...]
    ha = jnp.concatenate([d * (agg[c] + hsin[c]) for c in range(4)], axis=1)
    z = jax.nn.sigmoid(jnp.dot(ha, wz[...], preferred_element_type=jnp.float32) + bz[...])
    ht = jnp.tanh(jnp.dot(ha, wh[...], preferred_element_type=jnp.float32) + bh[...])
    out[...] = (1.0 - z) * ht


def _tc_layer2(agg2, hs, dis_b, wz, wh, bz, bh):
    grid = (_NP // 256,)
    return pl.pallas_call(
        _layer2_body,
        grid=grid,
        in_specs=[
            pl.BlockSpec((4, 256, 128), lambda i: (0, i, 0)),
            pl.BlockSpec((4, 256, 128), lambda i: (0, i, 0)),
            pl.BlockSpec((256, 128), lambda i: (i, 0)),
            pl.BlockSpec((_D_H, _D_H), lambda i: (0, 0)),
            pl.BlockSpec((_D_H, _D_H), lambda i: (0, 0)),
            pl.BlockSpec((1, _D_H), lambda i: (0, 0)),
            pl.BlockSpec((1, _D_H), lambda i: (0, 0)),
        ],
        out_specs=pl.BlockSpec((256, _D_H), lambda i: (i, 0)),
        out_shape=jax.ShapeDtypeStruct((_NP, _D_H), jnp.float32),
    )(agg2.reshape(4, _NP, 128), hs, dis_b, wz, wh, bz, bh)


def kernel(x, edge_index, conv1_W, conv1_b, lin1_W, lin1_b,
           conv2_W, conv2_b, lin2_W, lin2_b):
    src = edge_index[0]
    dst = edge_index[1]
    pad = _EP - _E
    src_p = jnp.concatenate([src, jnp.zeros((pad,), jnp.int32)]).reshape(_ER, _CH)
    dst_p = jnp.concatenate([dst, jnp.full((pad,), _N, jnp.int32)]).reshape(_ER, _CH)
    x_p = jnp.pad(x, ((0, _NP - _N), (0, 0)))

    wz1, wh1, wz2, wh2, bz1, bh1, bz2, bh2 = _combine_weights(
        conv1_W, conv1_b, lin1_W, lin1_b, conv2_W, conv2_b, lin2_W, lin2_b)

    zeros_c = jnp.zeros((_NP * 8,), jnp.float32)
    deg_part = _make_deg_kernel()(dst_p, zeros_c).reshape(32, _NP, 8)
    dis_b, xs = _tc_scale(deg_part, x_p)

    edges_il = jnp.stack([src_p, dst_p], axis=1).reshape(2 * _ER, _CH)

    agg1_raw = _make_agg_kernel(2)(edges_il, _pack_groups(xs, 2), zeros_c)
    agg1 = agg1_raw.reshape(2, 16, _NP, 8).transpose(0, 2, 1, 3).reshape(2 * _NP, 128)
    hs = _tc_layer1(agg1, xs, dis_b, wz1, wh1, bz1, bh1)

    agg2_raw = _make_agg_kernel(4)(edges_il, _pack_groups(hs, 4), zeros_c)
    agg2 = agg2_raw.reshape(4, 16, _NP, 8).transpose(0, 2, 1, 3).reshape(4 * _NP, 128)
    out_full = _tc_layer2(agg2, hs, dis_b, wz2, wh2, bz2, bh2)
    return out_full[:_N]


# speedup vs baseline: 1.4276x; 1.3173x over previous
import functools

import jax
import jax.numpy as jnp
from jax import lax
from jax.experimental import pallas as pl
from jax.experimental.pallas import tpu as pltpu
from jax.experimental.pallas import tpu_sc as plsc

_N = 10000
_E = 160000
_D_IN = 256
_D_H = 512

_NC, _NS = 2, 16
_NP = 10240
_CH = 128
_EP = 163840
_ER = _EP // _CH

@functools.cache
def _mesh():
    return plsc.VectorSubcoreMesh(core_axis_name="c", subcore_axis_name="s",
                                  num_cores=_NC, num_subcores=_NS)


@functools.cache
def _make_deg_kernel():
    return functools.partial(
        pl.kernel,
        out_type=jax.ShapeDtypeStruct((32, _NP * 8), jnp.float32),
        mesh=_mesh(),
        compiler_params=pltpu.CompilerParams(needs_layout_passes=False),
        scratch_types=[
            pltpu.VMEM((_ER // 32, _CH), jnp.int32),
            pltpu.VMEM((_NP * 8,), jnp.float32),
        ],
    )(_deg_body)


def _deg_body(dst_hbm, zeros_hbm, out_hbm, idx_v, acc_v):
    c = lax.axis_index("c")
    s = lax.axis_index("s")
    w = c * _NS + s
    pltpu.sync_copy(zeros_hbm, acc_v)
    pltpu.sync_copy(dst_hbm.at[pl.ds(w * 40, 40)], idx_v)
    ones16 = jnp.ones((16,), jnp.float32)

    def row(i, _):
        for j in range(8):
            d16 = idx_v[i, pl.ds(j * 16, 16)]
            plsc.addupdate_scatter(acc_v, [d16 * 8], ones16)
        return 0

    lax.fori_loop(0, 40, row, 0)
    pltpu.sync_copy(acc_v, out_hbm.at[w])


@functools.cache
def _make_agg_kernel(n_sides: int):
    ngpt = n_sides * 16 // 32
    SROWS = 8
    SEC = _ER // SROWS

    @functools.partial(
        pl.kernel,
        out_type=jax.ShapeDtypeStruct((32 * ngpt, _NP * 8), jnp.float32),
        mesh=_mesh(),
        compiler_params=pltpu.CompilerParams(needs_layout_passes=False),
        scratch_types=[
            pltpu.VMEM((2 * SROWS, _CH), jnp.int32),
            pltpu.VMEM((2 * SROWS, _CH), jnp.int32),
            pltpu.VMEM((_NP * 4,), jnp.int32),
            pltpu.VMEM((_NP * 8,), jnp.float32),
            pltpu.SemaphoreType.DMA,
            pltpu.SemaphoreType.DMA,
        ],
    )
    def _agg(edges_hbm, feat_hbm, zeros_hbm, out_hbm,
             b0, b1, feat_v, acc_v, sem0, sem1):
        c = lax.axis_index("c")
        s = lax.axis_index("s")
        w = c * _NS + s
        himask = jnp.full((16,), -65536, jnp.int32)
        iota16 = lax.iota(jnp.int32, 16)
        cp4 = iota16 % 4
        cp4x2 = cp4 * 2
        cp4x2p1 = cp4x2 + 1
        quad_pats = [iota16 // 4 + 4 * q for q in range(4)]

        def compute(buf):
            def rowbody(i, _):
                for k in range(8):
                    sl = pl.ds(k * 16, 16)
                    s4 = buf[2 * i, sl] * 4
                    d8 = buf[2 * i + 1, sl] * 8
                    for q in range(4):
                        pat = quad_pats[q]
                        squad = s4.at[pat].get(mode="promise_in_bounds")
                        dquad = d8.at[pat].get(mode="promise_in_bounds")
                        vals = plsc.load_gather(feat_v, [squad + cp4])
                        lo = plsc.bitcast(lax.shift_left(vals, 16),
                                          jnp.float32)
                        hi = plsc.bitcast(lax.bitwise_and(vals, himask),
                                          jnp.float32)
                        plsc.addupdate_scatter(acc_v, [dquad + cp4x2], lo)
                        plsc.addupdate_scatter(acc_v, [dquad + cp4x2p1], hi)
                return 0

            lax.fori_loop(0, SROWS, rowbody, 0)

        for gl in range(ngpt):
            g = w + 32 * gl
            pltpu.sync_copy(feat_hbm.at[g], feat_v)
            pltpu.sync_copy(zeros_hbm, acc_v)

            def body(sec, _):
                pltpu.sync_copy(
                    edges_hbm.at[pl.ds(sec * 2 * SROWS, 2 * SROWS)], b0)
                compute(b0)
                return 0

            lax.fori_loop(0, SEC, body, 0)
            pltpu.sync_copy(acc_v, out_hbm.at[g])

    return _agg


def _pack_groups(x, n_sides):
    t = x.reshape(n_sides, _NP, 16, 8).transpose(0, 2, 1, 3)
    t = t.astype(jnp.bfloat16).reshape(n_sides * 16, _NP, 4, 2)
    return lax.bitcast_convert_type(t, jnp.int32).reshape(n_sides * 16, _NP * 4)


def _combine_body(cw1z, cw1h, l1z, l1h, cw2z, cw2h, l2z, l2h,
                  cb1z, cb1h, lb1z, lb1h, cb2z, cb2h, lb2z, lb2h,
                  wz1, wh1, wz2, wh2, bz1, bh1, bz2, bh2):
    f32 = jnp.float32
    wz1[...] = jnp.dot(cw1z[...], l1z[...], preferred_element_type=f32)
    wh1[...] = jnp.dot(cw1h[...], l1h[...], preferred_element_type=f32)
    wz2[...] = jnp.dot(cw2z[...], l2z[...], preferred_element_type=f32)
    wh2[...] = jnp.dot(cw2h[...], l2h[...], preferred_element_type=f32)
    bz1[...] = jnp.dot(cb1z[...], l1z[...], preferred_element_type=f32) + lb1z[...]
    bh1[...] = jnp.dot(cb1h[...], l1h[...], preferred_element_type=f32) + lb1h[...]
    bz2[...] = jnp.dot(cb2z[...], l2z[...], preferred_element_type=f32) + lb2z[...]
    bh2[...] = jnp.dot(cb2h[...], l2h[...], preferred_element_type=f32) + lb2h[...]


def _combine_weights(conv1_W, conv1_b, lin1_W, lin1_b,
                     conv2_W, conv2_b, lin2_W, lin2_b):
    f32 = jnp.float32
    outs = [
        jax.ShapeDtypeStruct((_D_IN, _D_H), f32),
        jax.ShapeDtypeStruct((_D_IN, _D_H), f32),
        jax.ShapeDtypeStruct((_D_H, _D_H), f32),
        jax.ShapeDtypeStruct((_D_H, _D_H), f32),
        jax.ShapeDtypeStruct((1, _D_H), f32),
        jax.ShapeDtypeStruct((1, _D_H), f32),
        jax.ShapeDtypeStruct((1, _D_H), f32),
        jax.ShapeDtypeStruct((1, _D_H), f32),
    ]
    return pl.pallas_call(_combine_body, out_shape=outs)(
        conv1_W[0], conv1_W[2], lin1_W[0][:_D_H], lin1_W[2][:_D_H],
        conv2_W[0], conv2_W[2], lin2_W[0][:_D_H], lin2_W[2][:_D_H],
        conv1_b[0][None], conv1_b[2][None], lin1_b[0][None], lin1_b[2][None],
        conv2_b[0][None], conv2_b[2][None], lin2_b[0][None], lin2_b[2][None],
    )


def _scale_body(degp, xblk, disb, xs):
    deg = jnp.sum(degp[:, :, 0], axis=0) + 1.0
    dis = lax.rsqrt(deg)
    disb[...] = jnp.broadcast_to(dis[:, None], (256, 128))
    xsv = xblk[...] * dis[:, None]
    xs[0] = xsv[:, :128]
    xs[1] = xsv[:, 128:]


def _tc_scale(deg_part, x_p):
    grid = (_NP // 256,)
    return pl.pallas_call(
        _scale_body,
        grid=grid,
        in_specs=[
            pl.BlockSpec((32, 256, 8), lambda i: (0, i, 0)),
            pl.BlockSpec((256, _D_IN), lambda i: (i, 0)),
        ],
        out_specs=[
            pl.BlockSpec((256, 128), lambda i: (i, 0)),
            pl.BlockSpec((2, 256, 128), lambda i: (0, i, 0)),
        ],
        out_shape=[
            jax.ShapeDtypeStruct((_NP, 128), jnp.float32),
            jax.ShapeDtypeStruct((2, _NP, 128), jnp.float32),
        ],
    )(deg_part, x_p)


def _layer1_body(agg, xs, disb, wz, wh, bz, bh, hs):
    d = disb[...]
    xa = jnp.concatenate([d * (agg[0] + xs[0]), d * (agg[1] + xs[1])], axis=1)
    z = jax.nn.sigmoid(jnp.dot(xa, wz[...], preferred_element_type=jnp.float32) + bz[...])
    ht = jnp.tanh(jnp.dot(xa, wh[...], preferred_element_type=jnp.float32) + bh[...])
    h1 = jax.nn.relu((1.0 - z) * ht)
    for c in range(4):
        hs[c] = h1[:, c * 128:(c + 1) * 128] * d


def _tc_layer1(agg1, xs, dis_b, wz, wh, bz, bh):
    grid = (_NP // 256,)
    return pl.pallas_call(
        _layer1_body,
        grid=grid,
        in_specs=[
            pl.BlockSpec((2, 256, 128), lambda i: (0, i, 0)),
            pl.BlockSpec((2, 256, 128), lambda i: (0, i, 0)),
            pl.BlockSpec((256, 128), lambda i: (i, 0)),
            pl.BlockSpec((_D_IN, _D_H), lambda i: (0, 0)),
            pl.BlockSpec((_D_IN, _D_H), lambda i: (0, 0)),
            pl.BlockSpec((1, _D_H), lambda i: (0, 0)),
            pl.BlockSpec((1, _D_H), lambda i: (0, 0)),
        ],
        out_specs=pl.BlockSpec((4, 256, 128), lambda i: (0, i, 0)),
        out_shape=jax.ShapeDtypeStruct((4, _NP, 128), jnp.float32),
    )(agg1.reshape(2, _NP, 128), xs, dis_b, wz, wh, bz, bh)


def _layer2_body(agg, hsin, disb, wz, wh, bz, bh, out):
    d = disb[...]
    ha = jnp.concatenate([d * (agg[c] + hsin[c]) for c in range(4)], axis=1)
    z = jax.nn.sigmoid(jnp.dot(ha, wz[...], preferred_element_type=jnp.float32) + bz[...])
    ht = jnp.tanh(jnp.dot(ha, wh[...], preferred_element_type=jnp.float32) + bh[...])
    out[...] = (1.0 - z) * ht


def _tc_layer2(agg2, hs, dis_b, wz, wh, bz, bh):
    grid = (_NP // 256,)
    return pl.pallas_call(
        _layer2_body,
        grid=grid,
        in_specs=[
            pl.BlockSpec((4, 256, 128), lambda i: (0, i, 0)),
            pl.BlockSpec((4, 256, 128), lambda i: (0, i, 0)),
            pl.BlockSpec((256, 128), lambda i: (i, 0)),
            pl.BlockSpec((_D_H, _D_H), lambda i: (0, 0)),
            pl.BlockSpec((_D_H, _D_H), lambda i: (0, 0)),
            pl.BlockSpec((1, _D_H), lambda i: (0, 0)),
            pl.BlockSpec((1, _D_H), lambda i: (0, 0)),
        ],
        out_specs=pl.BlockSpec((256, _D_H), lambda i: (i, 0)),
        out_shape=jax.ShapeDtypeStruct((_NP, _D_H), jnp.float32),
    )(agg2.reshape(4, _NP, 128), hs, dis_b, wz, wh, bz, bh)


def kernel(x, edge_index, conv1_W, conv1_b, lin1_W, lin1_b,
           conv2_W, conv2_b, lin2_W, lin2_b):
    src = edge_index[0]
    dst = edge_index[1]
    pad = _EP - _E
    src_p = jnp.concatenate([src, jnp.zeros((pad,), jnp.int32)]).reshape(_ER, _CH)
    dst_p = jnp.concatenate([dst, jnp.full((pad,), _N, jnp.int32)]).reshape(_ER, _CH)
    x_p = jnp.pad(x, ((0, _NP - _N), (0, 0)))

    wz1, wh1, wz2, wh2, bz1, bh1, bz2, bh2 = _combine_weights(
        conv1_W, conv1_b, lin1_W, lin1_b, conv2_W, conv2_b, lin2_W, lin2_b)

    zeros_c = jnp.zeros((_NP * 8,), jnp.float32)
    deg_part = _make_deg_kernel()(dst_p, zeros_c).reshape(32, _NP, 8)
    dis_b, xs = _tc_scale(deg_part, x_p)

    edges_il = jnp.stack([src_p, dst_p], axis=1).reshape(2 * _ER, _CH)

    agg1_raw = _make_agg_kernel(2)(edges_il, _pack_groups(xs, 2), zeros_c)
    agg1 = agg1_raw.reshape(2, 16, _NP, 8).transpose(0, 2, 1, 3).reshape(2 * _NP, 128)
    hs = _tc_layer1(agg1, xs, dis_b, wz1, wh1, bz1, bh1)

    agg2_raw = _make_agg_kernel(4)(edges_il, _pack_groups(hs, 4), zeros_c)
    agg2 = agg2_raw.reshape(4, 16, _NP, 8).transpose(0, 2, 1, 3).reshape(4 * _NP, 128)
    out_full = _tc_layer2(agg2, hs, dis_b, wz2, wh2, bz2, bh2)
    return out_full[:_N]


# speedup vs baseline: 1.6402x; 1.1489x over previous
import functools

import jax
import jax.numpy as jnp
from jax import lax
from jax.experimental import pallas as pl
from jax.experimental.pallas import tpu as pltpu
from jax.experimental.pallas import tpu_sc as plsc

_N = 10000
_E = 160000
_D_IN = 256
_D_H = 512

_NC, _NS = 2, 16
_NP = 10240
_CH = 128
_EP = 163840
_ER = _EP // _CH

@functools.cache
def _mesh():
    return plsc.VectorSubcoreMesh(core_axis_name="c", subcore_axis_name="s",
                                  num_cores=_NC, num_subcores=_NS)


@functools.cache
def _make_deg_kernel():
    return functools.partial(
        pl.kernel,
        out_type=jax.ShapeDtypeStruct((32, _NP * 8), jnp.float32),
        mesh=_mesh(),
        compiler_params=pltpu.CompilerParams(needs_layout_passes=False),
        scratch_types=[
            pltpu.VMEM((_ER // 32, _CH), jnp.int32),
            pltpu.VMEM((_NP * 8,), jnp.float32),
        ],
    )(_deg_body)


def _deg_body(dst_hbm, zeros_hbm, out_hbm, idx_v, acc_v):
    c = lax.axis_index("c")
    s = lax.axis_index("s")
    w = c * _NS + s
    pltpu.sync_copy(zeros_hbm, acc_v)
    pltpu.sync_copy(dst_hbm.at[pl.ds(w * 40, 40)], idx_v)
    ones16 = jnp.ones((16,), jnp.float32)

    def row(i, _):
        for j in range(8):
            d16 = idx_v[i, pl.ds(j * 16, 16)]
            plsc.addupdate_scatter(acc_v, [d16 * 8], ones16)
        return 0

    lax.fori_loop(0, 40, row, 0)
    pltpu.sync_copy(acc_v, out_hbm.at[w])


@functools.cache
def _make_agg_kernel(n_sides: int):
    ngpt = n_sides * 16 // 32
    SROWS = 8
    SEC = _ER // SROWS

    @functools.partial(
        pl.kernel,
        out_type=jax.ShapeDtypeStruct((32 * ngpt, _NP * 8), jnp.float32),
        mesh=_mesh(),
        compiler_params=pltpu.CompilerParams(needs_layout_passes=False),
        scratch_types=[
            pltpu.VMEM((2 * SROWS, _CH), jnp.int32),
            pltpu.VMEM((2 * SROWS, _CH), jnp.int32),
            pltpu.VMEM((_NP * 4,), jnp.int32),
            pltpu.VMEM((_NP * 8,), jnp.float32),
            pltpu.SemaphoreType.DMA,
            pltpu.SemaphoreType.DMA,
        ],
    )
    def _agg(edges_hbm, feat_hbm, zeros_hbm, out_hbm,
             b0, b1, feat_v, acc_v, sem0, sem1):
        c = lax.axis_index("c")
        s = lax.axis_index("s")
        w = c * _NS + s
        himask = jnp.full((16,), -65536, jnp.int32)
        iota16 = lax.iota(jnp.int32, 16)
        cp4 = iota16 % 4
        cp4x2 = cp4 * 2
        cp4x2p1 = cp4x2 + 1
        quad_pats = [iota16 // 4 + 4 * q for q in range(4)]

        def compute(buf):
            def rowbody(i, _):
                for k in range(8):
                    sl = pl.ds(k * 16, 16)
                    s4 = buf[2 * i, sl] * 4
                    d8 = buf[2 * i + 1, sl] * 8
                    for q in range(4):
                        pat = quad_pats[q]
                        squad = s4.at[pat].get(mode="promise_in_bounds")
                        dquad = d8.at[pat].get(mode="promise_in_bounds")
                        vals = plsc.load_gather(feat_v, [squad + cp4])
                        lo = plsc.bitcast(lax.shift_left(vals, 16),
                                          jnp.float32)
                        hi = plsc.bitcast(lax.bitwise_and(vals, himask),
                                          jnp.float32)
                        plsc.addupdate_scatter(acc_v, [dquad + cp4x2], lo)
                        plsc.addupdate_scatter(acc_v, [dquad + cp4x2p1], hi)
                return 0

            lax.fori_loop(0, SROWS, rowbody, 0)

        for gl in range(ngpt):
            g = w + 32 * gl
            pltpu.sync_copy(feat_hbm.at[g], feat_v)
            pltpu.sync_copy(zeros_hbm, acc_v)
            pltpu.async_copy(edges_hbm.at[pl.ds(0, 2 * SROWS)], b0, sem0)

            def pairbody(ss, _):
                r0 = (2 * ss) * 2 * SROWS
                r1 = (2 * ss + 1) * 2 * SROWS
                rn = lax.rem(2 * ss + 2, SEC) * 2 * SROWS
                pltpu.make_async_copy(
                    edges_hbm.at[pl.ds(r0, 2 * SROWS)], b0, sem0).wait()
                d1 = pltpu.async_copy(
                    edges_hbm.at[pl.ds(r1, 2 * SROWS)], b1, sem1)
                compute(b0)
                d1.wait()
                pltpu.async_copy(edges_hbm.at[pl.ds(rn, 2 * SROWS)], b0, sem0)
                compute(b1)
                return 0

            lax.fori_loop(0, SEC // 2, pairbody, 0)
            pltpu.make_async_copy(
                edges_hbm.at[pl.ds(0, 2 * SROWS)], b0, sem0).wait()
            pltpu.sync_copy(acc_v, out_hbm.at[g])

    return _agg


def _pack_groups(x, n_sides):
    t = x.reshape(n_sides, _NP, 16, 8).transpose(0, 2, 1, 3)
    t = t.astype(jnp.bfloat16).reshape(n_sides * 16, _NP, 4, 2)
    return lax.bitcast_convert_type(t, jnp.int32).reshape(n_sides * 16, _NP * 4)


def _combine_body(cw1z, cw1h, l1z, l1h, cw2z, cw2h, l2z, l2h,
                  cb1z, cb1h, lb1z, lb1h, cb2z, cb2h, lb2z, lb2h,
                  wz1, wh1, wz2, wh2, bz1, bh1, bz2, bh2):
    f32 = jnp.float32
    wz1[...] = jnp.dot(cw1z[...], l1z[...], preferred_element_type=f32)
    wh1[...] = jnp.dot(cw1h[...], l1h[...], preferred_element_type=f32)
    wz2[...] = jnp.dot(cw2z[...], l2z[...], preferred_element_type=f32)
    wh2[...] = jnp.dot(cw2h[...], l2h[...], preferred_element_type=f32)
    bz1[...] = jnp.dot(cb1z[...], l1z[...], preferred_element_type=f32) + lb1z[...]
    bh1[...] = jnp.dot(cb1h[...], l1h[...], preferred_element_type=f32) + lb1h[...]
    bz2[...] = jnp.dot(cb2z[...], l2z[...], preferred_element_type=f32) + lb2z[...]
    bh2[...] = jnp.dot(cb2h[...], l2h[...], preferred_element_type=f32) + lb2h[...]


def _combine_weights(conv1_W, conv1_b, lin1_W, lin1_b,
                     conv2_W, conv2_b, lin2_W, lin2_b):
    f32 = jnp.float32
    outs = [
        jax.ShapeDtypeStruct((_D_IN, _D_H), f32),
        jax.ShapeDtypeStruct((_D_IN, _D_H), f32),
        jax.ShapeDtypeStruct((_D_H, _D_H), f32),
        jax.ShapeDtypeStruct((_D_H, _D_H), f32),
        jax.ShapeDtypeStruct((1, _D_H), f32),
        jax.ShapeDtypeStruct((1, _D_H), f32),
        jax.ShapeDtypeStruct((1, _D_H), f32),
        jax.ShapeDtypeStruct((1, _D_H), f32),
    ]
    return pl.pallas_call(_combine_body, out_shape=outs)(
        conv1_W[0], conv1_W[2], lin1_W[0][:_D_H], lin1_W[2][:_D_H],
        conv2_W[0], conv2_W[2], lin2_W[0][:_D_H], lin2_W[2][:_D_H],
        conv1_b[0][None], conv1_b[2][None], lin1_b[0][None], lin1_b[2][None],
        conv2_b[0][None], conv2_b[2][None], lin2_b[0][None], lin2_b[2][None],
    )


def _scale_body(degp, xblk, disb, xs):
    deg = jnp.sum(degp[:, :, 0], axis=0) + 1.0
    dis = lax.rsqrt(deg)
    disb[...] = jnp.broadcast_to(dis[:, None], (256, 128))
    xsv = xblk[...] * dis[:, None]
    xs[0] = xsv[:, :128]
    xs[1] = xsv[:, 128:]


def _tc_scale(deg_part, x_p):
    grid = (_NP // 256,)
    return pl.pallas_call(
        _scale_body,
        grid=grid,
        in_specs=[
            pl.BlockSpec((32, 256, 8), lambda i: (0, i, 0)),
            pl.BlockSpec((256, _D_IN), lambda i: (i, 0)),
        ],
        out_specs=[
            pl.BlockSpec((256, 128), lambda i: (i, 0)),
            pl.BlockSpec((2, 256, 128), lambda i: (0, i, 0)),
        ],
        out_shape=[
            jax.ShapeDtypeStruct((_NP, 128), jnp.float32),
            jax.ShapeDtypeStruct((2, _NP, 128), jnp.float32),
        ],
    )(deg_part, x_p)


def _layer1_body(agg, xs, disb, wz, wh, bz, bh, hs):
    d = disb[...]
    xa = jnp.concatenate([d * (agg[0] + xs[0]), d * (agg[1] + xs[1])], axis=1)
    z = jax.nn.sigmoid(jnp.dot(xa, wz[...], preferred_element_type=jnp.float32) + bz[...])
    ht = jnp.tanh(jnp.dot(xa, wh[...], preferred_element_type=jnp.float32) + bh[...])
    h1 = jax.nn.relu((1.0 - z) * ht)
    for c in range(4):
        hs[c] = h1[:, c * 128:(c + 1) * 128] * d


def _tc_layer1(agg1, xs, dis_b, wz, wh, bz, bh):
    grid = (_NP // 256,)
    return pl.pallas_call(
        _layer1_body,
        grid=grid,
        in_specs=[
            pl.BlockSpec((2, 256, 128), lambda i: (0, i, 0)),
            pl.BlockSpec((2, 256, 128), lambda i: (0, i, 0)),
            pl.BlockSpec((256, 128), lambda i: (i, 0)),
            pl.BlockSpec((_D_IN, _D_H), lambda i: (0, 0)),
            pl.BlockSpec((_D_IN, _D_H), lambda i: (0, 0)),
            pl.BlockSpec((1, _D_H), lambda i: (0, 0)),
            pl.BlockSpec((1, _D_H), lambda i: (0, 0)),
        ],
        out_specs=pl.BlockSpec((4, 256, 128), lambda i: (0, i, 0)),
        out_shape=jax.ShapeDtypeStruct((4, _NP, 128), jnp.float32),
    )(agg1.reshape(2, _NP, 128), xs, dis_b, wz, wh, bz, bh)


def _layer2_body(agg, hsin, disb, wz, wh, bz, bh, out):
    d = disb[...]
    ha = jnp.concatenate([d * (agg[c] + hsin[c]) for c in range(4)], axis=1)
    z = jax.nn.sigmoid(jnp.dot(ha, wz[...], preferred_element_type=jnp.float32) + bz[...])
    ht = jnp.tanh(jnp.dot(ha, wh[...], preferred_element_type=jnp.float32) + bh[...])
    out[...] = (1.0 - z) * ht


def _tc_layer2(agg2, hs, dis_b, wz, wh, bz, bh):
    grid = (_NP // 256,)
    return pl.pallas_call(
        _layer2_body,
        grid=grid,
        in_specs=[
            pl.BlockSpec((4, 256, 128), lambda i: (0, i, 0)),
            pl.BlockSpec((4, 256, 128), lambda i: (0, i, 0)),
            pl.BlockSpec((256, 128), lambda i: (i, 0)),
            pl.BlockSpec((_D_H, _D_H), lambda i: (0, 0)),
            pl.BlockSpec((_D_H, _D_H), lambda i: (0, 0)),
            pl.BlockSpec((1, _D_H), lambda i: (0, 0)),
            pl.BlockSpec((1, _D_H), lambda i: (0, 0)),
        ],
        out_specs=pl.BlockSpec((256, _D_H), lambda i: (i, 0)),
        out_shape=jax.ShapeDtypeStruct((_NP, _D_H), jnp.float32),
    )(agg2.reshape(4, _NP, 128), hs, dis_b, wz, wh, bz, bh)


def kernel(x, edge_index, conv1_W, conv1_b, lin1_W, lin1_b,
           conv2_W, conv2_b, lin2_W, lin2_b):
    src = edge_index[0]
    dst = edge_index[1]
    pad = _EP - _E
    src_p = jnp.concatenate([src, jnp.zeros((pad,), jnp.int32)]).reshape(_ER, _CH)
    dst_p = jnp.concatenate([dst, jnp.full((pad,), _N, jnp.int32)]).reshape(_ER, _CH)
    x_p = jnp.pad(x, ((0, _NP - _N), (0, 0)))

    wz1, wh1, wz2, wh2, bz1, bh1, bz2, bh2 = _combine_weights(
        conv1_W, conv1_b, lin1_W, lin1_b, conv2_W, conv2_b, lin2_W, lin2_b)

    zeros_c = jnp.zeros((_NP * 8,), jnp.float32)
    deg_part = _make_deg_kernel()(dst_p, zeros_c).reshape(32, _NP, 8)
    dis_b, xs = _tc_scale(deg_part, x_p)

    edges_il = jnp.stack([src_p, dst_p], axis=1).reshape(2 * _ER, _CH)

    agg1_raw = _make_agg_kernel(2)(edges_il, _pack_groups(xs, 2), zeros_c)
    agg1 = agg1_raw.reshape(2, 16, _NP, 8).transpose(0, 2, 1, 3).reshape(2 * _NP, 128)
    hs = _tc_layer1(agg1, xs, dis_b, wz1, wh1, bz1, bh1)

    agg2_raw = _make_agg_kernel(4)(edges_il, _pack_groups(hs, 4), zeros_c)
    agg2 = agg2_raw.reshape(4, 16, _NP, 8).transpose(0, 2, 1, 3).reshape(4 * _NP, 128)
    out_full = _tc_layer2(agg2, hs, dis_b, wz2, wh2, bz2, bh2)
    return out_full[:_N]
